# baseline (speedup 1.0000x reference)
import jax
import jax.numpy as jnp
from jax.experimental import pallas as pl
from jax.experimental.pallas import tpu as pltpu


def _vq_body(x_ref, w_ref, q_ref, idx_ref):
    x = x_ref[0]
    w = w_ref[...]
    K = w.shape[0]
    TT = x.shape[1]

    s2 = jax.lax.dot_general(
        -2.0 * w, x, (((1,), (0,)), ((), ())),
        preferred_element_type=jnp.float32)
    wn = jnp.sum(w * w, axis=1, keepdims=True)
    xn = jnp.sum(x * x, axis=0, keepdims=True)
    dist = (wn + xn) + s2

    iota = jax.lax.broadcasted_iota(jnp.int32, (K, TT), 0)
    m = jnp.min(dist, axis=0, keepdims=True)
    idx = jnp.min(jnp.where(dist == m, iota, K), axis=0, keepdims=True)

    oh = (iota == idx).astype(jnp.float32)
    q = jax.lax.dot_general(
        w, oh, (((0,), (0,)), ((), ())),
        preferred_element_type=jnp.float32)

    q_ref[0] = x + (q - x)
    idx_ref[0] = idx


_TT = 1024


def kernel(inputs, W):
    B, D, T = inputs.shape
    K = W.shape[0]
    nt = T // _TT
    q, idx = pl.pallas_call(
        _vq_body,
        grid=(B, nt),
        in_specs=[
            pl.BlockSpec((1, D, _TT), lambda b, j: (b, 0, j)),
            pl.BlockSpec((K, D), lambda b, j: (0, 0)),
        ],
        out_specs=[
            pl.BlockSpec((1, D, _TT), lambda b, j: (b, 0, j)),
            pl.BlockSpec((1, 1, _TT), lambda b, j: (b, 0, j)),
        ],
        out_shape=[
            jax.ShapeDtypeStruct((B, D, T), jnp.float32),
            jax.ShapeDtypeStruct((B, 1, T), jnp.int32),
        ],
        compiler_params=pltpu.CompilerParams(
            dimension_semantics=("parallel", "parallel")),
    )(inputs, W)
    return (q, idx.reshape(B * T, 1))


# speedup vs baseline: 1.0571x; 1.0571x over previous
import jax
import jax.numpy as jnp
from jax.experimental import pallas as pl
from jax.experimental.pallas import tpu as pltpu


def _vq_body(x_ref, w_ref, q_ref, idx_ref):
    x = x_ref[0]
    w = w_ref[...]
    K = w.shape[0]
    TT = x.shape[1]

    s2 = jax.lax.dot_general(
        -2.0 * w, x, (((1,), (0,)), ((), ())),
        preferred_element_type=jnp.float32)
    wn = jnp.sum(w * w, axis=1, keepdims=True)
    xn = jnp.sum(x * x, axis=0, keepdims=True)
    dist = (wn + xn) + s2

    fiota = jax.lax.broadcasted_iota(jnp.int32, (K, 1), 0).astype(jnp.float32)
    m = jnp.min(dist, axis=0, keepdims=True)
    fidx = jnp.min(jnp.where(dist == m, fiota, float(K)), axis=0,
                   keepdims=True)

    oh = jnp.where(fiota == fidx, 1.0, 0.0)
    idx = fidx.astype(jnp.int32)
    q = jax.lax.dot_general(
        w, oh, (((0,), (0,)), ((), ())),
        preferred_element_type=jnp.float32)

    q_ref[0] = x + (q - x)
    idx_ref[0] = idx


_TT = 1024


def kernel(inputs, W):
    B, D, T = inputs.shape
    K = W.shape[0]
    nt = T // _TT
    q, idx = pl.pallas_call(
        _vq_body,
        grid=(B, nt),
        in_specs=[
            pl.BlockSpec((1, D, _TT), lambda b, j: (b, 0, j)),
            pl.BlockSpec((K, D), lambda b, j: (0, 0)),
        ],
        out_specs=[
            pl.BlockSpec((1, D, _TT), lambda b, j: (b, 0, j)),
            pl.BlockSpec((1, 1, _TT), lambda b, j: (b, 0, j)),
        ],
        out_shape=[
            jax.ShapeDtypeStruct((B, D, T), jnp.float32),
            jax.ShapeDtypeStruct((B, 1, T), jnp.int32),
        ],
        compiler_params=pltpu.CompilerParams(
            dimension_semantics=("parallel", "parallel")),
    )(inputs, W)
    return (q, idx.reshape(B * T, 1))
